# baseline (speedup 1.0000x reference)
import jax
import jax.numpy as jnp
from jax import lax
from jax.experimental import pallas as pl
from jax.experimental.pallas import tpu as pltpu
from jax.experimental.pallas import tpu_sc as plsc

N = 10000
E = 160000
NSUB = 16
CH = 125
NCH = E // (NSUB * CH)
RPT = N // NSUB
BN = 1000


def _make_agg(d_half: int, want_deg: bool, edge_split: bool = False):
  mesh = plsc.VectorSubcoreMesh(
      core_axis_name="c", subcore_axis_name="s", num_cores=2,
      num_subcores=NSUB)
  outs = [jax.ShapeDtypeStruct((N, d_half), jnp.float32),
          jax.ShapeDtypeStruct((N, d_half), jnp.float32)]
  scratch = [
      pltpu.VMEM((NCH // 2, CH), jnp.int32),
      pltpu.VMEM((NCH // 2, CH), jnp.int32),
      pltpu.VMEM((CH, d_half), jnp.float32),
      pltpu.VMEM((CH, d_half), jnp.float32),
      pltpu.VMEM_SHARED((N, d_half), jnp.float32),
      pltpu.SemaphoreType.DMA,
      pltpu.SemaphoreType.DMA,
      pltpu.SemaphoreType.DMA,
      pltpu.SemaphoreType.DMA,
  ]
  if want_deg:
    outs.append(jax.ShapeDtypeStruct((N,), jnp.float32))
    scratch += [
        pltpu.VMEM((128,), jnp.float32),
        pltpu.VMEM((1008,), jnp.float32),
        pltpu.VMEM_SHARED((N,), jnp.float32),
    ]

  def body(*refs):
    if edge_split:
      (p_in, col_h, row_h, outL, outR,
       col_v, row_v, buf0, buf1, acc, sem0, sem1, ssem0, ssem1) = refs
      pL = pR = p_in
      deg_out = ones_v = zb = dega = None
    elif want_deg:
      (pL, pR, col_h, row_h, outL, outR, deg_out,
       col_v, row_v, buf0, buf1, acc, sem0, sem1, ssem0, ssem1, ones_v, zb,
       dega) = refs
    else:
      (pL, pR, col_h, row_h, outL, outR,
       col_v, row_v, buf0, buf1, acc, sem0, sem1, ssem0, ssem1) = refs
      deg_out = ones_v = zb = dega = None
    c = lax.axis_index("c")
    s = lax.axis_index("s")

    z16 = jnp.zeros((16,), jnp.float32)
    npg = d_half // 16

    def zfill(i, carry):
      buf0[i // npg, pl.ds((i % npg) * 16, 16)] = z16
      return carry

    lax.fori_loop(0, 40 * npg, zfill, 0)
    zbase = pl.multiple_of(s * 1000, 8)

    @pl.when(s < 10)
    def _():
      for k in range(25):
        pltpu.sync_copy(buf0.at[pl.ds(0, 40)], acc.at[pl.ds(zbase + k * 40, 40)])

    if want_deg:
      o16 = jnp.ones((16,), jnp.float32)

      def ofill(i, carry):
        ones_v[pl.ds(i * 16, 16)] = o16
        return carry

      lax.fori_loop(0, 8, ofill, 0)

      def zfill1(i, carry):
        zb[pl.ds(i * 16, 16)] = z16
        return carry

      lax.fori_loop(0, 63, zfill1, 0)

      @pl.when(jnp.logical_and(c == 0, s < 10))
      def _():
        pltpu.sync_copy(zb.at[pl.ds(0, 1000)], dega.at[pl.ds(zbase, 1000)])

    plsc.subcore_barrier()

    nphase = NCH // 2

    def run(p_h, do_deg, phase_bases):
      for pbase in phase_bases:
        pltpu.sync_copy(col_h.at[s, pl.ds(pbase, nphase)], col_v)
        pltpu.sync_copy(row_h.at[s, pl.ds(pbase, nphase)], row_v)
        pltpu.async_copy(p_h.at[col_v.at[0]], buf0, sem0)
        pltpu.async_copy(p_h.at[col_v.at[1]], buf1, sem1)

        def step(g, carry):
          j0 = g * 2
          j1 = j0 + 1
          pltpu.make_async_copy(p_h.at[col_v.at[j0]], buf0, sem0).wait()
          pltpu.async_copy(buf0, acc.at[row_v.at[j0]], ssem0, add=True)
          if do_deg:
            pltpu.sync_copy(ones_v.at[pl.ds(0, CH)], dega.at[row_v.at[j0]],
                            add=True)
          pltpu.make_async_copy(p_h.at[col_v.at[j1]], buf1, sem1).wait()
          pltpu.async_copy(buf1, acc.at[row_v.at[j1]], ssem1, add=True)
          if do_deg:
            pltpu.sync_copy(ones_v.at[pl.ds(0, CH)], dega.at[row_v.at[j1]],
                            add=True)
          pltpu.make_async_copy(buf0, acc.at[row_v.at[j0]], ssem0).wait()
          pltpu.async_copy(p_h.at[col_v.at[j0 + 2]], buf0, sem0)
          pltpu.make_async_copy(buf1, acc.at[row_v.at[j1]], ssem1).wait()
          pltpu.async_copy(p_h.at[col_v.at[j1 + 2]], buf1, sem1)
          return carry

        lax.fori_loop(0, nphase // 2 - 1, step, 0)
        jf0 = nphase - 2
        jf1 = nphase - 1
        pltpu.make_async_copy(p_h.at[col_v.at[jf0]], buf0, sem0).wait()
        pltpu.async_copy(buf0, acc.at[row_v.at[jf0]], ssem0, add=True)
        if do_deg:
          pltpu.sync_copy(ones_v.at[pl.ds(0, CH)], dega.at[row_v.at[jf0]],
                          add=True)
        pltpu.make_async_copy(p_h.at[col_v.at[jf1]], buf1, sem1).wait()
        pltpu.async_copy(buf1, acc.at[row_v.at[jf1]], ssem1, add=True)
        if do_deg:
          pltpu.sync_copy(ones_v.at[pl.ds(0, CH)], dega.at[row_v.at[jf1]],
                          add=True)
        pltpu.make_async_copy(buf0, acc.at[row_v.at[jf0]], ssem0).wait()
        pltpu.make_async_copy(buf1, acc.at[row_v.at[jf1]], ssem1).wait()

    if edge_split:
      run(pL, False, [pl.multiple_of(c * nphase, 8)])
    else:
      @pl.when(c == 0)
      def _():
        run(pL, want_deg, [0, nphase])

      @pl.when(c == 1)
      def _():
        run(pR, False, [0, nphase])

    plsc.subcore_barrier()

    @pl.when(jnp.logical_and(c == 0, s < 10))
    def _():
      pltpu.sync_copy(acc.at[pl.ds(zbase, 1000)],
                      outL.at[pl.ds(zbase, 1000)])
      if want_deg:
        pltpu.sync_copy(dega.at[pl.ds(zbase, 1000)], zb.at[pl.ds(0, 1000)])
        pltpu.sync_copy(zb.at[pl.ds(0, 1000)], deg_out.at[pl.ds(zbase, 1000)])

    @pl.when(jnp.logical_and(c == 1, s < 10))
    def _():
      pltpu.sync_copy(acc.at[pl.ds(zbase, 1000)],
                      outR.at[pl.ds(zbase, 1000)])

  return pl.kernel(body, out_type=tuple(outs), mesh=mesh,
                   scratch_types=scratch)


_agg_deg = _make_agg(128, True)
_agg128 = _make_agg(128, False)
_agg_edge = _make_agg(128, False, edge_split=True)


def _premul(x, wl, wr):
  d_in = x.shape[1]
  dh = wl.shape[1]

  def body(x_ref, wl_ref, wr_ref, oL_ref, oR_ref):
    xb = x_ref[...]
    oL_ref[...] = jnp.dot(xb, wl_ref[...], preferred_element_type=jnp.float32)
    oR_ref[...] = jnp.dot(xb, wr_ref[...], preferred_element_type=jnp.float32)

  return pl.pallas_call(
      body,
      grid=(N // BN,),
      in_specs=[pl.BlockSpec((BN, d_in), lambda i: (i, 0)),
                pl.BlockSpec((d_in, dh), lambda i: (0, 0)),
                pl.BlockSpec((d_in, dh), lambda i: (0, 0))],
      out_specs=[pl.BlockSpec((BN, dh), lambda i: (i, 0)),
                 pl.BlockSpec((BN, dh), lambda i: (i, 0))],
      out_shape=[jax.ShapeDtypeStruct((N, dh), jnp.float32),
                 jax.ShapeDtypeStruct((N, dh), jnp.float32)],
  )(x, wl, wr)


def _dense(aggL, aggR, deg2, h_in, r_mat, b2, *ws):
  dh = aggL.shape[1]
  d_in = h_in.shape[1]

  def body(*refs):
    aL, aR, dg, hi, r_ref, b_ref = refs[:6]
    w_refs = refs[6:6 + len(ws)]
    h_o = refs[6 + len(ws)]
    p_os = refs[7 + len(ws):]
    inv = 1.0 / jnp.maximum(dg[...], 1.0)
    mean = jnp.concatenate([aL[...], aR[...]], axis=1) * inv
    h = mean + jnp.dot(hi[...], r_ref[...],
                       preferred_element_type=jnp.float32) + b_ref[...]
    h = jnp.maximum(h, 0.0)
    h_o[...] = h
    for w_ref, p_o in zip(w_refs, p_os):
      p_o[...] = jnp.dot(h, w_ref[...], preferred_element_type=jnp.float32)

  return pl.pallas_call(
      body,
      grid=(N // BN,),
      in_specs=[pl.BlockSpec((BN, dh), lambda i: (i, 0)),
                pl.BlockSpec((BN, dh), lambda i: (i, 0)),
                pl.BlockSpec((BN, 1), lambda i: (i, 0)),
                pl.BlockSpec((BN, d_in), lambda i: (i, 0)),
                pl.BlockSpec((d_in, 2 * dh), lambda i: (0, 0)),
                pl.BlockSpec((1, 2 * dh), lambda i: (0, 0))]
               + [pl.BlockSpec(w.shape, lambda i: (0, 0)) for w in ws],
      out_specs=[pl.BlockSpec((BN, 2 * dh), lambda i: (i, 0))]
                + [pl.BlockSpec((BN, w.shape[1]), lambda i: (i, 0))
                   for w in ws],
      out_shape=[jax.ShapeDtypeStruct((N, 2 * dh), jnp.float32)]
                + [jax.ShapeDtypeStruct((N, w.shape[1]), jnp.float32)
                   for w in ws],
  )(aggL, aggR, deg2, h_in, r_mat, b2, *ws)


def _final(aggL, aggR, deg2, h_in, r_mat, b2):
  dh = aggL.shape[1]
  d_in = h_in.shape[1]
  d_out = r_mat.shape[1]

  def body(aL, aR, dg, hi, r_ref, b_ref, o_ref):
    inv = 1.0 / jnp.maximum(dg[...], 1.0)
    z = ((aL[...] + aR[...]) * inv
         + jnp.dot(hi[...], r_ref[...], preferred_element_type=jnp.float32)
         + b_ref[...])
    z = z - jnp.max(z, axis=1, keepdims=True)
    o_ref[...] = z - jnp.log(jnp.sum(jnp.exp(z), axis=1, keepdims=True))

  return pl.pallas_call(
      body,
      grid=(N // BN,),
      in_specs=[pl.BlockSpec((BN, dh), lambda i: (i, 0)),
                pl.BlockSpec((BN, dh), lambda i: (i, 0)),
                pl.BlockSpec((BN, 1), lambda i: (i, 0)),
                pl.BlockSpec((BN, d_in), lambda i: (i, 0)),
                pl.BlockSpec((d_in, d_out), lambda i: (0, 0)),
                pl.BlockSpec((1, d_out), lambda i: (0, 0))],
      out_specs=pl.BlockSpec((BN, d_out), lambda i: (i, 0)),
      out_shape=jax.ShapeDtypeStruct((N, d_out), jnp.float32),
  )(aggL, aggR, deg2, h_in, r_mat, b2)


def kernel(x, edge_index, W1, R1, b1, W2, R2, b2, W3, R3, b3):
  row = edge_index[0].reshape(NSUB, NCH, CH)
  col = edge_index[1].reshape(NSUB, NCH, CH)

  p1L, p1R = _premul(x, W1[:, :128], W1[:, 128:])
  aggL1, aggR1, deg = _agg_deg(p1L, p1R, col, row)
  deg2 = deg.reshape(N, 1)
  h1, p2L, p2R = _dense(aggL1, aggR1, deg2, x, R1, b1.reshape(1, -1),
                        W2[:, :128], W2[:, 128:])
  aggL2, aggR2 = _agg128(p2L, p2R, col, row)
  h2, p3 = _dense(aggL2, aggR2, deg2, h1, R2, b2.reshape(1, -1), W3)
  aggA3, aggB3 = _agg_edge(p3, col, row)
  return _final(aggA3, aggB3, deg2, h2, R3, b3.reshape(1, -1))


# speedup vs baseline: 1.2383x; 1.2383x over previous
import jax
import jax.numpy as jnp
from jax import lax
from jax.experimental import pallas as pl
from jax.experimental.pallas import tpu as pltpu
from jax.experimental.pallas import tpu_sc as plsc

N = 10000
E = 160000
NSUB = 16
CH = 125
NCH = E // (NSUB * CH)
RPT = N // NSUB
BN = 1000


def _make_agg(d_half: int, want_deg: bool, edge_split: bool = False):
  mesh = plsc.VectorSubcoreMesh(
      core_axis_name="c", subcore_axis_name="s", num_cores=2,
      num_subcores=NSUB)
  outs = [jax.ShapeDtypeStruct((N, d_half), jnp.float32),
          jax.ShapeDtypeStruct((N, d_half), jnp.float32)]
  scratch = [
      pltpu.VMEM((NCH // 2, CH), jnp.int32),
      pltpu.VMEM((NCH // 2, CH), jnp.int32),
      pltpu.VMEM((CH, d_half), jnp.float32),
      pltpu.VMEM((CH, d_half), jnp.float32),
      pltpu.VMEM_SHARED((N, d_half), jnp.float32),
      pltpu.SemaphoreType.DMA,
      pltpu.SemaphoreType.DMA,
  ]
  if want_deg:
    outs.append(jax.ShapeDtypeStruct((N,), jnp.float32))
    scratch += [
        pltpu.VMEM((128,), jnp.float32),
        pltpu.VMEM((1008,), jnp.float32),
        pltpu.VMEM_SHARED((N,), jnp.float32),
    ]

  def body(*refs):
    if edge_split:
      (p_in, col_h, row_h, outL, outR,
       col_v, row_v, buf0, buf1, acc, sem0, sem1) = refs
      pL = pR = p_in
      deg_out = ones_v = zb = dega = None
    elif want_deg:
      (pL, pR, col_h, row_h, outL, outR, deg_out,
       col_v, row_v, buf0, buf1, acc, sem0, sem1, ones_v, zb,
       dega) = refs
    else:
      (pL, pR, col_h, row_h, outL, outR,
       col_v, row_v, buf0, buf1, acc, sem0, sem1) = refs
      deg_out = ones_v = zb = dega = None
    c = lax.axis_index("c")
    s = lax.axis_index("s")

    z16 = jnp.zeros((16,), jnp.float32)
    npg = d_half // 16

    def zfill(i, carry):
      buf0[i // npg, pl.ds((i % npg) * 16, 16)] = z16
      return carry

    lax.fori_loop(0, 40 * npg, zfill, 0)
    zbase = pl.multiple_of(s * 1000, 8)

    @pl.when(s < 10)
    def _():
      for k in range(25):
        pltpu.sync_copy(buf0.at[pl.ds(0, 40)], acc.at[pl.ds(zbase + k * 40, 40)])

    if want_deg:
      o16 = jnp.ones((16,), jnp.float32)

      def ofill(i, carry):
        ones_v[pl.ds(i * 16, 16)] = o16
        return carry

      lax.fori_loop(0, 8, ofill, 0)

      def zfill1(i, carry):
        zb[pl.ds(i * 16, 16)] = z16
        return carry

      lax.fori_loop(0, 63, zfill1, 0)

      @pl.when(jnp.logical_and(c == 0, s < 10))
      def _():
        pltpu.sync_copy(zb.at[pl.ds(0, 1000)], dega.at[pl.ds(zbase, 1000)])

    plsc.subcore_barrier()

    nphase = NCH // 2

    def run(p_h, do_deg, phase_bases):
      for pbase in phase_bases:
        pltpu.sync_copy(col_h.at[s, pl.ds(pbase, nphase)], col_v)
        pltpu.sync_copy(row_h.at[s, pl.ds(pbase, nphase)], row_v)
        pltpu.async_copy(p_h.at[col_v.at[0]], buf0, sem0)

        def step(g, carry):
          j0 = g * 2
          j1 = j0 + 1
          pltpu.async_copy(p_h.at[col_v.at[j1]], buf1, sem1)
          pltpu.make_async_copy(p_h.at[col_v.at[j0]], buf0, sem0).wait()
          pltpu.sync_copy(buf0, acc.at[row_v.at[j0]], add=True)
          if do_deg:
            pltpu.sync_copy(ones_v.at[pl.ds(0, CH)], dega.at[row_v.at[j0]],
                            add=True)

          @pl.when(j1 + 1 < nphase)
          def _():
            pltpu.async_copy(p_h.at[col_v.at[j1 + 1]], buf0, sem0)

          pltpu.make_async_copy(p_h.at[col_v.at[j1]], buf1, sem1).wait()
          pltpu.sync_copy(buf1, acc.at[row_v.at[j1]], add=True)
          if do_deg:
            pltpu.sync_copy(ones_v.at[pl.ds(0, CH)], dega.at[row_v.at[j1]],
                            add=True)
          return carry

        lax.fori_loop(0, nphase // 2, step, 0)

    if edge_split:
      run(pL, False, [pl.multiple_of(c * nphase, 8)])
    else:
      @pl.when(c == 0)
      def _():
        run(pL, want_deg, [0, nphase])

      @pl.when(c == 1)
      def _():
        run(pR, False, [0, nphase])

    plsc.subcore_barrier()

    @pl.when(jnp.logical_and(c == 0, s < 10))
    def _():
      pltpu.sync_copy(acc.at[pl.ds(zbase, 1000)],
                      outL.at[pl.ds(zbase, 1000)])
      if want_deg:
        pltpu.sync_copy(dega.at[pl.ds(zbase, 1000)], zb.at[pl.ds(0, 1000)])
        pltpu.sync_copy(zb.at[pl.ds(0, 1000)], deg_out.at[pl.ds(zbase, 1000)])

    @pl.when(jnp.logical_and(c == 1, s < 10))
    def _():
      pltpu.sync_copy(acc.at[pl.ds(zbase, 1000)],
                      outR.at[pl.ds(zbase, 1000)])

  return pl.kernel(body, out_type=tuple(outs), mesh=mesh,
                   scratch_types=scratch)


_agg_deg = _make_agg(128, True)
_agg128 = _make_agg(128, False)
_agg_edge = _make_agg(128, False, edge_split=True)


def _premul(x, wl, wr):
  d_in = x.shape[1]
  dh = wl.shape[1]

  def body(x_ref, wl_ref, wr_ref, oL_ref, oR_ref):
    xb = x_ref[...]
    oL_ref[...] = jnp.dot(xb, wl_ref[...], preferred_element_type=jnp.float32)
    oR_ref[...] = jnp.dot(xb, wr_ref[...], preferred_element_type=jnp.float32)

  return pl.pallas_call(
      body,
      grid=(N // BN,),
      in_specs=[pl.BlockSpec((BN, d_in), lambda i: (i, 0)),
                pl.BlockSpec((d_in, dh), lambda i: (0, 0)),
                pl.BlockSpec((d_in, dh), lambda i: (0, 0))],
      out_specs=[pl.BlockSpec((BN, dh), lambda i: (i, 0)),
                 pl.BlockSpec((BN, dh), lambda i: (i, 0))],
      out_shape=[jax.ShapeDtypeStruct((N, dh), jnp.float32),
                 jax.ShapeDtypeStruct((N, dh), jnp.float32)],
  )(x, wl, wr)


def _matmul_bias(h_in, r_mat, b2):
  d_in = h_in.shape[1]
  d_out = r_mat.shape[1]

  def body(hi, r_ref, b_ref, o_ref):
    o_ref[...] = jnp.dot(hi[...], r_ref[...],
                         preferred_element_type=jnp.float32) + b_ref[...]

  return pl.pallas_call(
      body,
      grid=(N // BN,),
      in_specs=[pl.BlockSpec((BN, d_in), lambda i: (i, 0)),
                pl.BlockSpec((d_in, d_out), lambda i: (0, 0)),
                pl.BlockSpec((1, d_out), lambda i: (0, 0))],
      out_specs=pl.BlockSpec((BN, d_out), lambda i: (i, 0)),
      out_shape=jax.ShapeDtypeStruct((N, d_out), jnp.float32),
  )(h_in, r_mat, b2)


def _combine(aggL, aggR, deg2, t, *ws):
  dh = aggL.shape[1]

  def body(*refs):
    aL, aR, dg, t_ref = refs[:4]
    w_refs = refs[4:4 + len(ws)]
    h_o = refs[4 + len(ws)]
    p_os = refs[5 + len(ws):]
    inv = 1.0 / jnp.maximum(dg[...], 1.0)
    mean = jnp.concatenate([aL[...], aR[...]], axis=1) * inv
    h = jnp.maximum(mean + t_ref[...], 0.0)
    h_o[...] = h
    for w_ref, p_o in zip(w_refs, p_os):
      p_o[...] = jnp.dot(h, w_ref[...], preferred_element_type=jnp.float32)

  return pl.pallas_call(
      body,
      grid=(N // BN,),
      in_specs=[pl.BlockSpec((BN, dh), lambda i: (i, 0)),
                pl.BlockSpec((BN, dh), lambda i: (i, 0)),
                pl.BlockSpec((BN, 1), lambda i: (i, 0)),
                pl.BlockSpec((BN, 2 * dh), lambda i: (i, 0))]
               + [pl.BlockSpec(w.shape, lambda i: (0, 0)) for w in ws],
      out_specs=[pl.BlockSpec((BN, 2 * dh), lambda i: (i, 0))]
                + [pl.BlockSpec((BN, w.shape[1]), lambda i: (i, 0))
                   for w in ws],
      out_shape=[jax.ShapeDtypeStruct((N, 2 * dh), jnp.float32)]
                + [jax.ShapeDtypeStruct((N, w.shape[1]), jnp.float32)
                   for w in ws],
  )(aggL, aggR, deg2, t, *ws)


def _final(aggA, aggB, deg2, t):
  dh = aggA.shape[1]

  def body(aA, aB, dg, t_ref, o_ref):
    inv = 1.0 / jnp.maximum(dg[...], 1.0)
    z = (aA[...] + aB[...]) * inv + t_ref[...]
    z = z - jnp.max(z, axis=1, keepdims=True)
    o_ref[...] = z - jnp.log(jnp.sum(jnp.exp(z), axis=1, keepdims=True))

  return pl.pallas_call(
      body,
      grid=(N // BN,),
      in_specs=[pl.BlockSpec((BN, dh), lambda i: (i, 0)),
                pl.BlockSpec((BN, dh), lambda i: (i, 0)),
                pl.BlockSpec((BN, 1), lambda i: (i, 0)),
                pl.BlockSpec((BN, dh), lambda i: (i, 0))],
      out_specs=pl.BlockSpec((BN, dh), lambda i: (i, 0)),
      out_shape=jax.ShapeDtypeStruct((N, dh), jnp.float32),
  )(aggA, aggB, deg2, t)


def kernel(x, edge_index, W1, R1, b1, W2, R2, b2, W3, R3, b3):
  row = edge_index[0].reshape(NSUB, NCH, CH)
  col = edge_index[1].reshape(NSUB, NCH, CH)

  p1L, p1R = _premul(x, W1[:, :128], W1[:, 128:])
  aggL1, aggR1, deg = _agg_deg(p1L, p1R, col, row)
  t1 = _matmul_bias(x, R1, b1.reshape(1, -1))
  deg2 = deg.reshape(N, 1)
  h1, p2L, p2R = _combine(aggL1, aggR1, deg2, t1, W2[:, :128], W2[:, 128:])
  aggL2, aggR2 = _agg128(p2L, p2R, col, row)
  t2 = _matmul_bias(h1, R2, b2.reshape(1, -1))
  h2, p3 = _combine(aggL2, aggR2, deg2, t2, W3)
  aggA3, aggB3 = _agg_edge(p3, col, row)
  t3 = _matmul_bias(h2, R3, b3.reshape(1, -1))
  return _final(aggA3, aggB3, deg2, t3)


# speedup vs baseline: 1.2386x; 1.0003x over previous
import jax
import jax.numpy as jnp
from jax import lax
from jax.experimental import pallas as pl
from jax.experimental.pallas import tpu as pltpu
from jax.experimental.pallas import tpu_sc as plsc

N = 10000
E = 160000
NSUB = 16
CH = 125
NCH = E // (NSUB * CH)
RPT = N // NSUB
BN = 1000


def _make_agg(d_half: int, want_deg: bool, edge_split: bool = False):
  mesh = plsc.VectorSubcoreMesh(
      core_axis_name="c", subcore_axis_name="s", num_cores=2,
      num_subcores=NSUB)
  outs = [jax.ShapeDtypeStruct((N, d_half), jnp.float32),
          jax.ShapeDtypeStruct((N, d_half), jnp.float32)]
  scratch = [
      pltpu.VMEM((NCH // 2, CH), jnp.int32),
      pltpu.VMEM((NCH // 2, CH), jnp.int32),
      pltpu.VMEM((CH, d_half), jnp.float32),
      pltpu.VMEM((CH, d_half), jnp.float32),
      pltpu.VMEM_SHARED((N, d_half), jnp.float32),
      pltpu.SemaphoreType.DMA,
      pltpu.SemaphoreType.DMA,
  ]
  if want_deg:
    outs.append(jax.ShapeDtypeStruct((N,), jnp.float32))
    scratch += [
        pltpu.VMEM((128,), jnp.float32),
        pltpu.VMEM((1008,), jnp.float32),
        pltpu.VMEM_SHARED((N,), jnp.float32),
    ]

  def body(*refs):
    if edge_split:
      (p_in, col_h, row_h, outL, outR,
       col_v, row_v, buf0, buf1, acc, sem0, sem1) = refs
      pL = pR = p_in
      deg_out = ones_v = zb = dega = None
    elif want_deg:
      (pL, pR, col_h, row_h, outL, outR, deg_out,
       col_v, row_v, buf0, buf1, acc, sem0, sem1, ones_v, zb,
       dega) = refs
    else:
      (pL, pR, col_h, row_h, outL, outR,
       col_v, row_v, buf0, buf1, acc, sem0, sem1) = refs
      deg_out = ones_v = zb = dega = None
    c = lax.axis_index("c")
    s = lax.axis_index("s")

    z16 = jnp.zeros((16,), jnp.float32)
    npg = d_half // 16

    def zfill(i, carry):
      buf0[i // npg, pl.ds((i % npg) * 16, 16)] = z16
      return carry

    lax.fori_loop(0, 40 * npg, zfill, 0)
    zbase = pl.multiple_of(s * 1000, 8)

    @pl.when(s < 10)
    def _():
      for k in range(25):
        pltpu.sync_copy(buf0.at[pl.ds(0, 40)], acc.at[pl.ds(zbase + k * 40, 40)])

    if want_deg:
      o16 = jnp.ones((16,), jnp.float32)

      def ofill(i, carry):
        ones_v[pl.ds(i * 16, 16)] = o16
        return carry

      lax.fori_loop(0, 8, ofill, 0)

      def zfill1(i, carry):
        zb[pl.ds(i * 16, 16)] = z16
        return carry

      lax.fori_loop(0, 63, zfill1, 0)

      @pl.when(jnp.logical_and(c == 0, s < 10))
      def _():
        pltpu.sync_copy(zb.at[pl.ds(0, 1000)], dega.at[pl.ds(zbase, 1000)])

    plsc.subcore_barrier()

    nphase = NCH // 2

    def run(p_h, do_deg, phase_bases):
      for pbase in phase_bases:
        pltpu.sync_copy(col_h.at[s, pl.ds(pbase, nphase)], col_v)
        pltpu.sync_copy(row_h.at[s, pl.ds(pbase, nphase)], row_v)
        pltpu.async_copy(p_h.at[col_v.at[0]], buf0, sem0)

        def step(g, carry):
          j0 = g * 2
          j1 = j0 + 1
          pltpu.async_copy(p_h.at[col_v.at[j1]], buf1, sem1)
          pltpu.make_async_copy(p_h.at[col_v.at[j0]], buf0, sem0).wait()
          pltpu.sync_copy(buf0, acc.at[row_v.at[j0]], add=True)
          if do_deg:
            pltpu.sync_copy(ones_v.at[pl.ds(0, CH)], dega.at[row_v.at[j0]],
                            add=True)

          @pl.when(j1 + 1 < nphase)
          def _():
            pltpu.async_copy(p_h.at[col_v.at[j1 + 1]], buf0, sem0)

          pltpu.make_async_copy(p_h.at[col_v.at[j1]], buf1, sem1).wait()
          pltpu.sync_copy(buf1, acc.at[row_v.at[j1]], add=True)
          if do_deg:
            pltpu.sync_copy(ones_v.at[pl.ds(0, CH)], dega.at[row_v.at[j1]],
                            add=True)
          return carry

        lax.fori_loop(0, nphase // 2, step, 0)

    if edge_split:
      run(pL, False, [pl.multiple_of(c * nphase, 8)])
    else:
      @pl.when(c == 0)
      def _():
        run(pL, want_deg, [0, nphase])

      @pl.when(c == 1)
      def _():
        run(pR, False, [0, nphase])

    plsc.subcore_barrier()

    @pl.when(jnp.logical_and(c == 0, s < 10))
    def _():
      pltpu.sync_copy(acc.at[pl.ds(zbase, 1000)],
                      outL.at[pl.ds(zbase, 1000)])
      if want_deg:
        pltpu.sync_copy(dega.at[pl.ds(zbase, 1000)], zb.at[pl.ds(0, 1000)])
        pltpu.sync_copy(zb.at[pl.ds(0, 1000)], deg_out.at[pl.ds(zbase, 1000)])

    @pl.when(jnp.logical_and(c == 1, s < 10))
    def _():
      pltpu.sync_copy(acc.at[pl.ds(zbase, 1000)],
                      outR.at[pl.ds(zbase, 1000)])

  return pl.kernel(body, out_type=tuple(outs), mesh=mesh,
                   scratch_types=scratch)


_agg_deg = _make_agg(128, True)
_agg128 = _make_agg(128, False)
_agg_edge = _make_agg(128, False, edge_split=True)


def _premul(x, wl, wr, r_mat, b2):
  d_in = x.shape[1]
  dh = wl.shape[1]
  d_r = r_mat.shape[1]

  def body(x_ref, wl_ref, wr_ref, r_ref, b_ref, oL_ref, oR_ref, t_ref):
    xb = x_ref[...]
    oL_ref[...] = jnp.dot(xb, wl_ref[...], preferred_element_type=jnp.float32)
    oR_ref[...] = jnp.dot(xb, wr_ref[...], preferred_element_type=jnp.float32)
    t_ref[...] = jnp.dot(xb, r_ref[...],
                         preferred_element_type=jnp.float32) + b_ref[...]

  return pl.pallas_call(
      body,
      grid=(N // BN,),
      in_specs=[pl.BlockSpec((BN, d_in), lambda i: (i, 0)),
                pl.BlockSpec((d_in, dh), lambda i: (0, 0)),
                pl.BlockSpec((d_in, dh), lambda i: (0, 0)),
                pl.BlockSpec((d_in, d_r), lambda i: (0, 0)),
                pl.BlockSpec((1, d_r), lambda i: (0, 0))],
      out_specs=[pl.BlockSpec((BN, dh), lambda i: (i, 0)),
                 pl.BlockSpec((BN, dh), lambda i: (i, 0)),
                 pl.BlockSpec((BN, d_r), lambda i: (i, 0))],
      out_shape=[jax.ShapeDtypeStruct((N, dh), jnp.float32),
                 jax.ShapeDtypeStruct((N, dh), jnp.float32),
                 jax.ShapeDtypeStruct((N, d_r), jnp.float32)],
  )(x, wl, wr, r_mat, b2)


def _combine(aggL, aggR, deg2, t, r_next, b_next, *ws):
  dh = aggL.shape[1]
  d_r = r_next.shape[1]

  def body(*refs):
    aL, aR, dg, t_ref, r_ref, b_ref = refs[:6]
    w_refs = refs[6:6 + len(ws)]
    p_os = refs[6 + len(ws):6 + 2 * len(ws)]
    tn_o = refs[6 + 2 * len(ws)]
    inv = 1.0 / jnp.maximum(dg[...], 1.0)
    mean = jnp.concatenate([aL[...], aR[...]], axis=1) * inv
    h = jnp.maximum(mean + t_ref[...], 0.0)
    for w_ref, p_o in zip(w_refs, p_os):
      p_o[...] = jnp.dot(h, w_ref[...], preferred_element_type=jnp.float32)
    tn_o[...] = jnp.dot(h, r_ref[...],
                        preferred_element_type=jnp.float32) + b_ref[...]

  return pl.pallas_call(
      body,
      grid=(N // BN,),
      in_specs=[pl.BlockSpec((BN, dh), lambda i: (i, 0)),
                pl.BlockSpec((BN, dh), lambda i: (i, 0)),
                pl.BlockSpec((BN, 1), lambda i: (i, 0)),
                pl.BlockSpec((BN, 2 * dh), lambda i: (i, 0)),
                pl.BlockSpec((2 * dh, d_r), lambda i: (0, 0)),
                pl.BlockSpec((1, d_r), lambda i: (0, 0))]
               + [pl.BlockSpec(w.shape, lambda i: (0, 0)) for w in ws],
      out_specs=[pl.BlockSpec((BN, w.shape[1]), lambda i: (i, 0))
                 for w in ws]
                + [pl.BlockSpec((BN, d_r), lambda i: (i, 0))],
      out_shape=[jax.ShapeDtypeStruct((N, w.shape[1]), jnp.float32)
                 for w in ws]
                + [jax.ShapeDtypeStruct((N, d_r), jnp.float32)],
  )(aggL, aggR, deg2, t, r_next, b_next, *ws)


def _final(aggA, aggB, deg2, t):
  dh = aggA.shape[1]

  def body(aA, aB, dg, t_ref, o_ref):
    inv = 1.0 / jnp.maximum(dg[...], 1.0)
    z = (aA[...] + aB[...]) * inv + t_ref[...]
    z = z - jnp.max(z, axis=1, keepdims=True)
    o_ref[...] = z - jnp.log(jnp.sum(jnp.exp(z), axis=1, keepdims=True))

  return pl.pallas_call(
      body,
      grid=(N // BN,),
      in_specs=[pl.BlockSpec((BN, dh), lambda i: (i, 0)),
                pl.BlockSpec((BN, dh), lambda i: (i, 0)),
                pl.BlockSpec((BN, 1), lambda i: (i, 0)),
                pl.BlockSpec((BN, dh), lambda i: (i, 0))],
      out_specs=pl.BlockSpec((BN, dh), lambda i: (i, 0)),
      out_shape=jax.ShapeDtypeStruct((N, dh), jnp.float32),
  )(aggA, aggB, deg2, t)


def kernel(x, edge_index, W1, R1, b1, W2, R2, b2, W3, R3, b3):
  row = edge_index[0].reshape(NSUB, NCH, CH)
  col = edge_index[1].reshape(NSUB, NCH, CH)

  p1L, p1R, t1 = _premul(x, W1[:, :128], W1[:, 128:], R1, b1.reshape(1, -1))
  aggL1, aggR1, deg = _agg_deg(p1L, p1R, col, row)
  deg2 = deg.reshape(N, 1)
  p2L, p2R, t2 = _combine(aggL1, aggR1, deg2, t1, R2, b2.reshape(1, -1),
                          W2[:, :128], W2[:, 128:])
  aggL2, aggR2 = _agg128(p2L, p2R, col, row)
  p3, t3 = _combine(aggL2, aggR2, deg2, t2, R3, b3.reshape(1, -1), W3)
  aggA3, aggB3 = _agg_edge(p3, col, row)
  return _final(aggA3, aggB3, deg2, t3)


# speedup vs baseline: 1.2492x; 1.0085x over previous
import jax
import jax.numpy as jnp
from jax import lax
from jax.experimental import pallas as pl
from jax.experimental.pallas import tpu as pltpu
from jax.experimental.pallas import tpu_sc as plsc

N = 10000
E = 160000
NSUB = 16
CH = 125
NCH = E // (NSUB * CH)
RPT = N // NSUB
BN = 1000


def _make_agg(d_half: int, want_deg: bool, edge_split: bool = False):
  mesh = plsc.VectorSubcoreMesh(
      core_axis_name="c", subcore_axis_name="s", num_cores=2,
      num_subcores=NSUB)
  outs = [jax.ShapeDtypeStruct((N, d_half), jnp.float32),
          jax.ShapeDtypeStruct((N, d_half), jnp.float32)]
  scratch = [
      pltpu.VMEM((NCH // 2, CH), jnp.int32),
      pltpu.VMEM((NCH // 2, CH), jnp.int32),
      pltpu.VMEM((CH, d_half), jnp.float32),
      pltpu.VMEM((CH, d_half), jnp.float32),
      pltpu.VMEM_SHARED((N, d_half), jnp.float32),
      pltpu.SemaphoreType.DMA,
      pltpu.SemaphoreType.DMA,
  ]
  if want_deg:
    outs.append(jax.ShapeDtypeStruct((N,), jnp.float32))
    scratch += [
        pltpu.VMEM((128,), jnp.float32),
        pltpu.VMEM((1008,), jnp.float32),
        pltpu.VMEM_SHARED((N,), jnp.float32),
    ]

  def body(*refs):
    if edge_split:
      (p_in, col_h, row_h, outL, outR,
       col_v, row_v, buf0, buf1, acc, sem0, sem1) = refs
      pL = pR = p_in
      deg_out = ones_v = zb = dega = None
    elif want_deg:
      (pL, pR, col_h, row_h, outL, outR, deg_out,
       col_v, row_v, buf0, buf1, acc, sem0, sem1, ones_v, zb,
       dega) = refs
    else:
      (pL, pR, col_h, row_h, outL, outR,
       col_v, row_v, buf0, buf1, acc, sem0, sem1) = refs
      deg_out = ones_v = zb = dega = None
    c = lax.axis_index("c")
    s = lax.axis_index("s")

    z16 = jnp.zeros((16,), jnp.float32)
    npg = d_half // 16

    def zfill(i, carry):
      buf0[i // npg, pl.ds((i % npg) * 16, 16)] = z16
      return carry

    lax.fori_loop(0, 40 * npg, zfill, 0)
    zbase = pl.multiple_of(s * 1000, 8)

    @pl.when(s < 10)
    def _():
      for k in range(25):
        pltpu.async_copy(buf0.at[pl.ds(0, 40)],
                         acc.at[pl.ds(zbase + k * 40, 40)], sem0)
      for k in range(25):
        pltpu.make_async_copy(buf0.at[pl.ds(0, 40)],
                              acc.at[pl.ds(zbase + k * 40, 40)], sem0).wait()

    if want_deg:
      o16 = jnp.ones((16,), jnp.float32)

      def ofill(i, carry):
        ones_v[pl.ds(i * 16, 16)] = o16
        return carry

      lax.fori_loop(0, 8, ofill, 0)

      def zfill1(i, carry):
        zb[pl.ds(i * 16, 16)] = z16
        return carry

      lax.fori_loop(0, 63, zfill1, 0)

      @pl.when(jnp.logical_and(c == 0, s < 10))
      def _():
        pltpu.sync_copy(zb.at[pl.ds(0, 1000)], dega.at[pl.ds(zbase, 1000)])

    plsc.subcore_barrier()

    nphase = NCH // 2

    def run(p_h, do_deg, phase_bases):
      for pbase in phase_bases:
        pltpu.sync_copy(col_h.at[s, pl.ds(pbase, nphase)], col_v)
        pltpu.sync_copy(row_h.at[s, pl.ds(pbase, nphase)], row_v)
        pltpu.async_copy(p_h.at[col_v.at[0]], buf0, sem0)

        def step(g, carry):
          j0 = g * 2
          j1 = j0 + 1
          pltpu.async_copy(p_h.at[col_v.at[j1]], buf1, sem1)
          pltpu.make_async_copy(p_h.at[col_v.at[j0]], buf0, sem0).wait()
          pltpu.sync_copy(buf0, acc.at[row_v.at[j0]], add=True)
          if do_deg:
            pltpu.sync_copy(ones_v.at[pl.ds(0, CH)], dega.at[row_v.at[j0]],
                            add=True)

          @pl.when(j1 + 1 < nphase)
          def _():
            pltpu.async_copy(p_h.at[col_v.at[j1 + 1]], buf0, sem0)

          pltpu.make_async_copy(p_h.at[col_v.at[j1]], buf1, sem1).wait()
          pltpu.sync_copy(buf1, acc.at[row_v.at[j1]], add=True)
          if do_deg:
            pltpu.sync_copy(ones_v.at[pl.ds(0, CH)], dega.at[row_v.at[j1]],
                            add=True)
          return carry

        lax.fori_loop(0, nphase // 2, step, 0)

    if edge_split:
      run(pL, False, [pl.multiple_of(c * nphase, 8)])
    else:
      @pl.when(c == 0)
      def _():
        run(pL, want_deg, [0, nphase])

      @pl.when(c == 1)
      def _():
        run(pR, False, [0, nphase])

    plsc.subcore_barrier()

    @pl.when(jnp.logical_and(c == 0, s < 10))
    def _():
      pltpu.sync_copy(acc.at[pl.ds(zbase, 1000)],
                      outL.at[pl.ds(zbase, 1000)])
      if want_deg:
        pltpu.sync_copy(dega.at[pl.ds(zbase, 1000)], zb.at[pl.ds(0, 1000)])
        pltpu.sync_copy(zb.at[pl.ds(0, 1000)], deg_out.at[pl.ds(zbase, 1000)])

    @pl.when(jnp.logical_and(c == 1, s < 10))
    def _():
      pltpu.sync_copy(acc.at[pl.ds(zbase, 1000)],
                      outR.at[pl.ds(zbase, 1000)])

  return pl.kernel(body, out_type=tuple(outs), mesh=mesh,
                   scratch_types=scratch)


_agg_deg = _make_agg(128, True)
_agg128 = _make_agg(128, False)
_agg_edge = _make_agg(128, False, edge_split=True)


def _premul(x, wl, wr, r_mat, b2):
  d_in = x.shape[1]
  dh = wl.shape[1]
  d_r = r_mat.shape[1]

  def body(x_ref, wl_ref, wr_ref, r_ref, b_ref, oL_ref, oR_ref, t_ref):
    xb = x_ref[...]
    oL_ref[...] = jnp.dot(xb, wl_ref[...], preferred_element_type=jnp.float32)
    oR_ref[...] = jnp.dot(xb, wr_ref[...], preferred_element_type=jnp.float32)
    t_ref[...] = jnp.dot(xb, r_ref[...],
                         preferred_element_type=jnp.float32) + b_ref[...]

  return pl.pallas_call(
      body,
      grid=(N // BN,),
      in_specs=[pl.BlockSpec((BN, d_in), lambda i: (i, 0)),
                pl.BlockSpec((d_in, dh), lambda i: (0, 0)),
                pl.BlockSpec((d_in, dh), lambda i: (0, 0)),
                pl.BlockSpec((d_in, d_r), lambda i: (0, 0)),
                pl.BlockSpec((1, d_r), lambda i: (0, 0))],
      out_specs=[pl.BlockSpec((BN, dh), lambda i: (i, 0)),
                 pl.BlockSpec((BN, dh), lambda i: (i, 0)),
                 pl.BlockSpec((BN, d_r), lambda i: (i, 0))],
      out_shape=[jax.ShapeDtypeStruct((N, dh), jnp.float32),
                 jax.ShapeDtypeStruct((N, dh), jnp.float32),
                 jax.ShapeDtypeStruct((N, d_r), jnp.float32)],
  )(x, wl, wr, r_mat, b2)


def _combine(aggL, aggR, deg2, t, r_next, b_next, *ws):
  dh = aggL.shape[1]
  d_r = r_next.shape[1]

  def body(*refs):
    aL, aR, dg, t_ref, r_ref, b_ref = refs[:6]
    w_refs = refs[6:6 + len(ws)]
    p_os = refs[6 + len(ws):6 + 2 * len(ws)]
    tn_o = refs[6 + 2 * len(ws)]
    inv = 1.0 / jnp.maximum(dg[...], 1.0)
    mean = jnp.concatenate([aL[...], aR[...]], axis=1) * inv
    h = jnp.maximum(mean + t_ref[...], 0.0)
    for w_ref, p_o in zip(w_refs, p_os):
      p_o[...] = jnp.dot(h, w_ref[...], preferred_element_type=jnp.float32)
    tn_o[...] = jnp.dot(h, r_ref[...],
                        preferred_element_type=jnp.float32) + b_ref[...]

  return pl.pallas_call(
      body,
      grid=(N // BN,),
      in_specs=[pl.BlockSpec((BN, dh), lambda i: (i, 0)),
                pl.BlockSpec((BN, dh), lambda i: (i, 0)),
                pl.BlockSpec((BN, 1), lambda i: (i, 0)),
                pl.BlockSpec((BN, 2 * dh), lambda i: (i, 0)),
                pl.BlockSpec((2 * dh, d_r), lambda i: (0, 0)),
                pl.BlockSpec((1, d_r), lambda i: (0, 0))]
               + [pl.BlockSpec(w.shape, lambda i: (0, 0)) for w in ws],
      out_specs=[pl.BlockSpec((BN, w.shape[1]), lambda i: (i, 0))
                 for w in ws]
                + [pl.BlockSpec((BN, d_r), lambda i: (i, 0))],
      out_shape=[jax.ShapeDtypeStruct((N, w.shape[1]), jnp.float32)
                 for w in ws]
                + [jax.ShapeDtypeStruct((N, d_r), jnp.float32)],
  )(aggL, aggR, deg2, t, r_next, b_next, *ws)


def _final(aggA, aggB, deg2, t):
  dh = aggA.shape[1]

  def body(aA, aB, dg, t_ref, o_ref):
    inv = 1.0 / jnp.maximum(dg[...], 1.0)
    z = (aA[...] + aB[...]) * inv + t_ref[...]
    z = z - jnp.max(z, axis=1, keepdims=True)
    o_ref[...] = z - jnp.log(jnp.sum(jnp.exp(z), axis=1, keepdims=True))

  return pl.pallas_call(
      body,
      grid=(N // BN,),
      in_specs=[pl.BlockSpec((BN, dh), lambda i: (i, 0)),
                pl.BlockSpec((BN, dh), lambda i: (i, 0)),
                pl.BlockSpec((BN, 1), lambda i: (i, 0)),
                pl.BlockSpec((BN, dh), lambda i: (i, 0))],
      out_specs=pl.BlockSpec((BN, dh), lambda i: (i, 0)),
      out_shape=jax.ShapeDtypeStruct((N, dh), jnp.float32),
  )(aggA, aggB, deg2, t)


def kernel(x, edge_index, W1, R1, b1, W2, R2, b2, W3, R3, b3):
  row = edge_index[0].reshape(NSUB, NCH, CH)
  col = edge_index[1].reshape(NSUB, NCH, CH)

  p1L, p1R, t1 = _premul(x, W1[:, :128], W1[:, 128:], R1, b1.reshape(1, -1))
  aggL1, aggR1, deg = _agg_deg(p1L, p1R, col, row)
  deg2 = deg.reshape(N, 1)
  p2L, p2R, t2 = _combine(aggL1, aggR1, deg2, t1, R2, b2.reshape(1, -1),
                          W2[:, :128], W2[:, 128:])
  aggL2, aggR2 = _agg128(p2L, p2R, col, row)
  p3, t3 = _combine(aggL2, aggR2, deg2, t2, R3, b3.reshape(1, -1), W3)
  aggA3, aggB3 = _agg_edge(p3, col, row)
  return _final(aggA3, aggB3, deg2, t3)


# speedup vs baseline: 1.2571x; 1.0063x over previous
import jax
import jax.numpy as jnp
from jax import lax
from jax.experimental import pallas as pl
from jax.experimental.pallas import tpu as pltpu
from jax.experimental.pallas import tpu_sc as plsc

N = 10000
E = 160000
NSUB = 16
CH = 125
NCH = E // (NSUB * CH)
RPT = N // NSUB
BN = 1000


def _make_agg(d_half: int, want_deg: bool, edge_split: bool = False):
  mesh = plsc.VectorSubcoreMesh(
      core_axis_name="c", subcore_axis_name="s", num_cores=2,
      num_subcores=NSUB)
  outs = [jax.ShapeDtypeStruct((N, d_half), jnp.float32),
          jax.ShapeDtypeStruct((N, d_half), jnp.float32)]
  scratch = [
      pltpu.VMEM((NCH // 2, CH), jnp.int32),
      pltpu.VMEM((NCH // 2, CH), jnp.int32),
      pltpu.VMEM((CH, d_half), jnp.float32),
      pltpu.VMEM((CH, d_half), jnp.float32),
      pltpu.VMEM_SHARED((N, d_half), jnp.float32),
      pltpu.SemaphoreType.DMA,
      pltpu.SemaphoreType.DMA,
  ]
  if want_deg:
    outs.append(jax.ShapeDtypeStruct((N,), jnp.float32))
    scratch += [
        pltpu.VMEM((128,), jnp.float32),
        pltpu.VMEM((1008,), jnp.float32),
        pltpu.VMEM_SHARED((N,), jnp.float32),
    ]

  def body(*refs):
    if edge_split:
      (p_in, col_h, row_h, outL, outR,
       col_v, row_v, buf0, buf1, acc, sem0, sem1) = refs
      pL = pR = p_in
      deg_out = ones_v = zb = dega = None
    elif want_deg:
      (pL, pR, col_h, row_h, outL, outR, deg_out,
       col_v, row_v, buf0, buf1, acc, sem0, sem1, ones_v, zb,
       dega) = refs
    else:
      (pL, pR, col_h, row_h, outL, outR,
       col_v, row_v, buf0, buf1, acc, sem0, sem1) = refs
      deg_out = ones_v = zb = dega = None
    c = lax.axis_index("c")
    s = lax.axis_index("s")
    nphase = NCH // 2

    pb0 = pl.multiple_of(c * nphase, 8) if edge_split else 0
    pltpu.sync_copy(col_h.at[s, pl.ds(pb0, nphase)], col_v)
    pltpu.sync_copy(row_h.at[s, pl.ds(pb0, nphase)], row_v)

    z16 = jnp.zeros((16,), jnp.float32)
    npg = d_half // 16

    def zfill(i, carry):
      buf0[i // npg, pl.ds((i % npg) * 16, 16)] = z16
      return carry

    lax.fori_loop(0, 40 * npg, zfill, 0)
    zbase = pl.multiple_of(s * 1000, 8)

    @pl.when(s < 10)
    def _():
      for k in range(25):
        pltpu.async_copy(buf0.at[pl.ds(0, 40)],
                         acc.at[pl.ds(zbase + k * 40, 40)], sem0)
      for k in range(25):
        pltpu.make_async_copy(buf0.at[pl.ds(0, 40)],
                              acc.at[pl.ds(zbase + k * 40, 40)], sem0).wait()

    if want_deg:
      o16 = jnp.ones((16,), jnp.float32)

      def ofill(i, carry):
        ones_v[pl.ds(i * 16, 16)] = o16
        return carry

      lax.fori_loop(0, 8, ofill, 0)

      def zfill1(i, carry):
        zb[pl.ds(i * 16, 16)] = z16
        return carry

      lax.fori_loop(0, 63, zfill1, 0)

      @pl.when(jnp.logical_and(c == 0, s < 10))
      def _():
        pltpu.sync_copy(zb.at[pl.ds(0, 1000)], dega.at[pl.ds(zbase, 1000)])

    plsc.subcore_barrier()

    def run(p_h, do_deg, phase_bases):
      for i, pbase in enumerate(phase_bases):
        if i > 0:
          pltpu.sync_copy(col_h.at[s, pl.ds(pbase, nphase)], col_v)
        pltpu.async_copy(p_h.at[col_v.at[0]], buf0, sem0)
        if i > 0:
          pltpu.sync_copy(row_h.at[s, pl.ds(pbase, nphase)], row_v)

        def step(g, carry):
          j0 = g * 2
          j1 = j0 + 1
          pltpu.async_copy(p_h.at[col_v.at[j1]], buf1, sem1)
          pltpu.make_async_copy(p_h.at[col_v.at[j0]], buf0, sem0).wait()
          pltpu.sync_copy(buf0, acc.at[row_v.at[j0]], add=True)
          if do_deg:
            pltpu.sync_copy(ones_v.at[pl.ds(0, CH)], dega.at[row_v.at[j0]],
                            add=True)

          @pl.when(j1 + 1 < nphase)
          def _():
            pltpu.async_copy(p_h.at[col_v.at[j1 + 1]], buf0, sem0)

          pltpu.make_async_copy(p_h.at[col_v.at[j1]], buf1, sem1).wait()
          pltpu.sync_copy(buf1, acc.at[row_v.at[j1]], add=True)
          if do_deg:
            pltpu.sync_copy(ones_v.at[pl.ds(0, CH)], dega.at[row_v.at[j1]],
                            add=True)
          return carry

        lax.fori_loop(0, nphase // 2, step, 0)

    if edge_split:
      run(pL, False, [pl.multiple_of(c * nphase, 8)])
    else:
      @pl.when(c == 0)
      def _():
        run(pL, want_deg, [0, nphase])

      @pl.when(c == 1)
      def _():
        run(pR, False, [0, nphase])

    plsc.subcore_barrier()

    @pl.when(jnp.logical_and(c == 0, s < 10))
    def _():
      pltpu.sync_copy(acc.at[pl.ds(zbase, 1000)],
                      outL.at[pl.ds(zbase, 1000)])
      if want_deg:
        pltpu.sync_copy(dega.at[pl.ds(zbase, 1000)], zb.at[pl.ds(0, 1000)])
        pltpu.sync_copy(zb.at[pl.ds(0, 1000)], deg_out.at[pl.ds(zbase, 1000)])

    @pl.when(jnp.logical_and(c == 1, s < 10))
    def _():
      pltpu.sync_copy(acc.at[pl.ds(zbase, 1000)],
                      outR.at[pl.ds(zbase, 1000)])

  return pl.kernel(body, out_type=tuple(outs), mesh=mesh,
                   scratch_types=scratch)


_agg_deg = _make_agg(128, True)
_agg128 = _make_agg(128, False)
_agg_edge = _make_agg(128, False, edge_split=True)


def _premul(x, wl, wr, r_mat, b2):
  d_in = x.shape[1]
  dh = wl.shape[1]
  d_r = r_mat.shape[1]

  def body(x_ref, wl_ref, wr_ref, r_ref, b_ref, oL_ref, oR_ref, t_ref):
    xb = x_ref[...]
    oL_ref[...] = jnp.dot(xb, wl_ref[...], preferred_element_type=jnp.float32)
    oR_ref[...] = jnp.dot(xb, wr_ref[...], preferred_element_type=jnp.float32)
    t_ref[...] = jnp.dot(xb, r_ref[...],
                         preferred_element_type=jnp.float32) + b_ref[...]

  return pl.pallas_call(
      body,
      grid=(N // BN,),
      in_specs=[pl.BlockSpec((BN, d_in), lambda i: (i, 0)),
                pl.BlockSpec((d_in, dh), lambda i: (0, 0)),
                pl.BlockSpec((d_in, dh), lambda i: (0, 0)),
                pl.BlockSpec((d_in, d_r), lambda i: (0, 0)),
                pl.BlockSpec((1, d_r), lambda i: (0, 0))],
      out_specs=[pl.BlockSpec((BN, dh), lambda i: (i, 0)),
                 pl.BlockSpec((BN, dh), lambda i: (i, 0)),
                 pl.BlockSpec((BN, d_r), lambda i: (i, 0))],
      out_shape=[jax.ShapeDtypeStruct((N, dh), jnp.float32),
                 jax.ShapeDtypeStruct((N, dh), jnp.float32),
                 jax.ShapeDtypeStruct((N, d_r), jnp.float32)],
  )(x, wl, wr, r_mat, b2)


def _combine(aggL, aggR, deg2, t, r_next, b_next, *ws):
  dh = aggL.shape[1]
  d_r = r_next.shape[1]

  def body(*refs):
    aL, aR, dg, t_ref, r_ref, b_ref = refs[:6]
    w_refs = refs[6:6 + len(ws)]
    p_os = refs[6 + len(ws):6 + 2 * len(ws)]
    tn_o = refs[6 + 2 * len(ws)]
    inv = 1.0 / jnp.maximum(dg[...], 1.0)
    mean = jnp.concatenate([aL[...], aR[...]], axis=1) * inv
    h = jnp.maximum(mean + t_ref[...], 0.0)
    for w_ref, p_o in zip(w_refs, p_os):
      p_o[...] = jnp.dot(h, w_ref[...], preferred_element_type=jnp.float32)
    tn_o[...] = jnp.dot(h, r_ref[...],
                        preferred_element_type=jnp.float32) + b_ref[...]

  return pl.pallas_call(
      body,
      grid=(N // BN,),
      in_specs=[pl.BlockSpec((BN, dh), lambda i: (i, 0)),
                pl.BlockSpec((BN, dh), lambda i: (i, 0)),
                pl.BlockSpec((BN, 1), lambda i: (i, 0)),
                pl.BlockSpec((BN, 2 * dh), lambda i: (i, 0)),
                pl.BlockSpec((2 * dh, d_r), lambda i: (0, 0)),
                pl.BlockSpec((1, d_r), lambda i: (0, 0))]
               + [pl.BlockSpec(w.shape, lambda i: (0, 0)) for w in ws],
      out_specs=[pl.BlockSpec((BN, w.shape[1]), lambda i: (i, 0))
                 for w in ws]
                + [pl.BlockSpec((BN, d_r), lambda i: (i, 0))],
      out_shape=[jax.ShapeDtypeStruct((N, w.shape[1]), jnp.float32)
                 for w in ws]
                + [jax.ShapeDtypeStruct((N, d_r), jnp.float32)],
  )(aggL, aggR, deg2, t, r_next, b_next, *ws)


def _final(aggA, aggB, deg2, t):
  dh = aggA.shape[1]

  def body(aA, aB, dg, t_ref, o_ref):
    inv = 1.0 / jnp.maximum(dg[...], 1.0)
    z = (aA[...] + aB[...]) * inv + t_ref[...]
    z = z - jnp.max(z, axis=1, keepdims=True)
    o_ref[...] = z - jnp.log(jnp.sum(jnp.exp(z), axis=1, keepdims=True))

  return pl.pallas_call(
      body,
      grid=(N // BN,),
      in_specs=[pl.BlockSpec((BN, dh), lambda i: (i, 0)),
                pl.BlockSpec((BN, dh), lambda i: (i, 0)),
                pl.BlockSpec((BN, 1), lambda i: (i, 0)),
                pl.BlockSpec((BN, dh), lambda i: (i, 0))],
      out_specs=pl.BlockSpec((BN, dh), lambda i: (i, 0)),
      out_shape=jax.ShapeDtypeStruct((N, dh), jnp.float32),
  )(aggA, aggB, deg2, t)


def kernel(x, edge_index, W1, R1, b1, W2, R2, b2, W3, R3, b3):
  row = edge_index[0].reshape(NSUB, NCH, CH)
  col = edge_index[1].reshape(NSUB, NCH, CH)

  p1L, p1R, t1 = _premul(x, W1[:, :128], W1[:, 128:], R1, b1.reshape(1, -1))
  aggL1, aggR1, deg = _agg_deg(p1L, p1R, col, row)
  deg2 = deg.reshape(N, 1)
  p2L, p2R, t2 = _combine(aggL1, aggR1, deg2, t1, R2, b2.reshape(1, -1),
                          W2[:, :128], W2[:, 128:])
  aggL2, aggR2 = _agg128(p2L, p2R, col, row)
  p3, t3 = _combine(aggL2, aggR2, deg2, t2, R3, b3.reshape(1, -1), W3)
  aggA3, aggB3 = _agg_edge(p3, col, row)
  return _final(aggA3, aggB3, deg2, t3)
